# baseline (speedup 1.0000x reference)
import functools

import jax
import jax.numpy as jnp
from jax import lax
from jax.experimental import pallas as pl
from jax.experimental.pallas import tpu as pltpu
from jax.experimental.pallas import tpu_sc as plsc

GRID = 32
NCELL = GRID ** 3
NPTS = 8192
NBATCH = 16
L = 16
NGROUPS = NPTS // L
NWORKERS = 32


def _affine_params(plane, quat):
    eye = jnp.eye(3, dtype=jnp.float32)
    n = plane[:, :3]
    n = n / (jnp.linalg.norm(n, axis=1, keepdims=True) + 1e-8)
    d = plane[:, 3:4]
    mp = eye[None] - 2.0 * n[:, :, None] * n[:, None, :]
    op = -2.0 * d * n
    q = quat / (jnp.linalg.norm(quat, axis=1, keepdims=True) + 1e-8)
    w = q[:, 0]
    qv = q[:, 1:]
    s2 = jnp.sum(qv * qv, axis=1)
    zero = jnp.zeros_like(w)
    kx, ky, kz = qv[:, 0], qv[:, 1], qv[:, 2]
    skew = jnp.stack([
        jnp.stack([zero, -kz, ky], axis=1),
        jnp.stack([kz, zero, -kx], axis=1),
        jnp.stack([-ky, kx, zero], axis=1),
    ], axis=1)
    mq = ((1.0 - 2.0 * s2)[:, None, None] * eye[None]
          + 2.0 * qv[:, :, None] * qv[:, None, :]
          + 2.0 * w[:, None, None] * skew)
    oq = jnp.zeros((quat.shape[0], 3), jnp.float32)
    mats = jnp.concatenate([mp, mq], axis=0).reshape(4, 9)
    offs = jnp.concatenate([op, oq], axis=0)
    return jnp.concatenate([mats, offs], axis=1)


def _pack_pair(a, b):
    au = lax.bitcast_convert_type(a.astype(jnp.bfloat16), jnp.uint16)
    bu = lax.bitcast_convert_type(b.astype(jnp.bfloat16), jnp.uint16)
    word = au.astype(jnp.uint32) | (bu.astype(jnp.uint32) << 16)
    return lax.bitcast_convert_type(word, jnp.int32)


@functools.cache
def _make_kernel():
    mesh = plsc.VectorSubcoreMesh(core_axis_name="c", subcore_axis_name="s")

    @functools.partial(
        pl.kernel,
        mesh=mesh,
        out_type=jax.ShapeDtypeStruct((2, NWORKERS, L), jnp.float32),
        scratch_types=[
            pltpu.VMEM((3, NPTS), jnp.float32),
            pltpu.VMEM((NPTS,), jnp.int32),
            pltpu.VMEM((NPTS,), jnp.int32),
            pltpu.VMEM((NPTS,), jnp.int32),
            pltpu.VMEM((NPTS,), jnp.int32),
            pltpu.VMEM((NPTS,), jnp.int32),
            pltpu.VMEM((NPTS,), jnp.int32),
            pltpu.VMEM((4, 12, L), jnp.float32),
            pltpu.VMEM((L,), jnp.float32),
            pltpu.SemaphoreType.DMA,
            pltpu.SemaphoreType.DMA,
        ],
    )
    def sym_loss_kernel(ptsx_hbm, taba_hbm, tabb_hbm, par_hbm, out_hbm,
                        pts_v, idx0_v, idx1_v,
                        ga0_v, gb0_v, ga1_v, gb1_v, par_v, stage_v,
                        sem0, sem1):
        c = lax.axis_index("c")
        s = lax.axis_index("s")
        wid = s * 2 + c
        t = wid // NBATCH
        b = wid % NBATCH
        pltpu.sync_copy(ptsx_hbm.at[b], pts_v)
        pltpu.sync_copy(par_hbm, par_v)
        base = b * NCELL
        idx_vs = [idx0_v, idx1_v]
        ga_vs = [ga0_v, ga1_v]
        gb_vs = [gb0_v, gb1_v]
        sems = [sem0, sem1]
        copies = [None, None]

        def transform_of(g):
            row = 2 * g + t
            return [par_v[row, k, :] for k in range(12)]

        def index_pass(g):
            m = transform_of(g)
            idx_v = idx_vs[g]

            def body_idx(i, carry):
                off = i * L
                x = pts_v[0, pl.ds(off, L)]
                y = pts_v[1, pl.ds(off, L)]
                z = pts_v[2, pl.ds(off, L)]
                tx = m[0] * x + m[1] * y + m[2] * z + m[9]
                ty = m[3] * x + m[4] * y + m[5] * z + m[10]
                tz = m[6] * x + m[7] * y + m[8] * z + m[11]
                ux = jnp.clip((tx + 0.5 - 0.015625) * 32.0, 0.0, 31.0)
                uy = jnp.clip((ty + 0.5 - 0.015625) * 32.0, 0.0, 31.0)
                uz = jnp.clip((tz + 0.5 - 0.015625) * 32.0, 0.0, 31.0)
                ix = (ux + 0.5).astype(jnp.int32)
                iy = (uy + 0.5).astype(jnp.int32)
                iz = (uz + 0.5).astype(jnp.int32)
                flat = ix * (GRID * GRID) + iy * GRID + iz
                idx_v[pl.ds(off, L)] = flat + base
                return carry

            lax.fori_loop(0, NGROUPS, body_idx, 0, unroll=4)
            copies[g] = (
                pltpu.async_copy(taba_hbm.at[idx_v], ga_vs[g], sems[g]),
                pltpu.async_copy(tabb_hbm.at[idx_v], gb_vs[g], sems[g]),
            )

        def acc_pass(g):
            m = transform_of(g)
            ga_v, gb_v = ga_vs[g], gb_vs[g]
            copies[g][0].wait()
            copies[g][1].wait()

            def body_acc(i, acc):
                off = i * L
                x = pts_v[0, pl.ds(off, L)]
                y = pts_v[1, pl.ds(off, L)]
                z = pts_v[2, pl.ds(off, L)]
                tx = m[0] * x + m[1] * y + m[2] * z + m[9]
                ty = m[3] * x + m[4] * y + m[5] * z + m[10]
                tz = m[6] * x + m[7] * y + m[8] * z + m[11]
                wa = ga_v[pl.ds(off, L)]
                wb = gb_v[pl.ds(off, L)]
                cx = lax.bitcast_convert_type(wa << 16, jnp.float32)
                cy = lax.bitcast_convert_type(wa & -65536, jnp.float32)
                cz = lax.bitcast_convert_type(wb << 16, jnp.float32)
                msk = lax.bitcast_convert_type(wb & -65536, jnp.float32)
                dx = tx - cx
                dy = ty - cy
                dz = tz - cz
                return acc + (dx * dx + dy * dy + dz * dz) * msk

            acc = lax.fori_loop(0, NGROUPS, body_acc,
                                jnp.zeros((L,), jnp.float32), unroll=4)
            stage_v[...] = acc
            pltpu.sync_copy(stage_v, out_hbm.at[g, wid])

        index_pass(0)
        index_pass(1)
        acc_pass(0)
        acc_pass(1)

    return sym_loss_kernel


def kernel(points, cp, voxel, plane, quat):
    pts_t = points.transpose(0, 2, 1)
    mask = 1.0 - voxel.reshape(NBATCH, NCELL)
    cp_t = cp.transpose(2, 0, 1)
    taba = _pack_pair(cp_t[0], cp_t[1]).reshape(NBATCH * NCELL)
    tabb = _pack_pair(cp_t[2], mask).reshape(NBATCH * NCELL)
    par = jnp.broadcast_to(_affine_params(plane, quat)[:, :, None],
                           (4, 12, L))
    parts = _make_kernel()(pts_t, taba, tabb, par)
    ref_loss = jnp.sum(parts[0]) / NBATCH
    rot_loss = jnp.sum(parts[1]) / NBATCH
    return (ref_loss, rot_loss)


# speedup vs baseline: 1.5400x; 1.5400x over previous
import functools

import jax
import jax.numpy as jnp
from jax import lax
from jax.experimental import pallas as pl
from jax.experimental.pallas import tpu as pltpu
from jax.experimental.pallas import tpu_sc as plsc

GRID = 32
NCELL = GRID ** 3
NPTS = 8192
NBATCH = 16
L = 16
NGROUPS = NPTS // L
NWORKERS = 32


def _affine_params(plane, quat):
    eye = jnp.eye(3, dtype=jnp.float32)
    n = plane[:, :3]
    n = n / (jnp.linalg.norm(n, axis=1, keepdims=True) + 1e-8)
    d = plane[:, 3:4]
    mp = eye[None] - 2.0 * n[:, :, None] * n[:, None, :]
    op = -2.0 * d * n
    q = quat / (jnp.linalg.norm(quat, axis=1, keepdims=True) + 1e-8)
    w = q[:, 0]
    qv = q[:, 1:]
    s2 = jnp.sum(qv * qv, axis=1)
    zero = jnp.zeros_like(w)
    kx, ky, kz = qv[:, 0], qv[:, 1], qv[:, 2]
    skew = jnp.stack([
        jnp.stack([zero, -kz, ky], axis=1),
        jnp.stack([kz, zero, -kx], axis=1),
        jnp.stack([-ky, kx, zero], axis=1),
    ], axis=1)
    mq = ((1.0 - 2.0 * s2)[:, None, None] * eye[None]
          + 2.0 * qv[:, :, None] * qv[:, None, :]
          + 2.0 * w[:, None, None] * skew)
    oq = jnp.zeros((quat.shape[0], 3), jnp.float32)
    mats = jnp.concatenate([mp, mq], axis=0).reshape(4, 9)
    offs = jnp.concatenate([op, oq], axis=0)
    return jnp.concatenate([mats, offs], axis=1)


def _pack_pair(a, b):
    au = lax.bitcast_convert_type(a.astype(jnp.bfloat16), jnp.uint16)
    bu = lax.bitcast_convert_type(b.astype(jnp.bfloat16), jnp.uint16)
    word = au.astype(jnp.uint32) | (bu.astype(jnp.uint32) << 16)
    return lax.bitcast_convert_type(word, jnp.int32)


@functools.cache
def _make_kernel():
    mesh = plsc.VectorSubcoreMesh(core_axis_name="c", subcore_axis_name="s")

    @functools.partial(
        pl.kernel,
        mesh=mesh,
        out_type=jax.ShapeDtypeStruct((2, NWORKERS, L), jnp.float32),
        scratch_types=[
            pltpu.VMEM((3, NPTS), jnp.float32),
            pltpu.VMEM((NPTS,), jnp.int32),
            pltpu.VMEM((NPTS,), jnp.int32),
            pltpu.VMEM((NPTS,), jnp.int32),
            pltpu.VMEM((NPTS,), jnp.int32),
            pltpu.VMEM((4, 12, L), jnp.float32),
            pltpu.VMEM((L,), jnp.float32),
            pltpu.SemaphoreType.DMA,
            pltpu.SemaphoreType.DMA,
        ],
    )
    def sym_loss_kernel(ptsx_hbm, tab_hbm, par_hbm, out_hbm,
                        pts_v, idx0_v, idx1_v,
                        ga0_v, ga1_v, par_v, stage_v,
                        sem0, sem1):
        c = lax.axis_index("c")
        s = lax.axis_index("s")
        wid = s * 2 + c
        t = wid // NBATCH
        b = wid % NBATCH
        pltpu.sync_copy(ptsx_hbm.at[b], pts_v)
        pltpu.sync_copy(par_hbm, par_v)
        base = b * NCELL
        idx_vs = [idx0_v, idx1_v]
        ga_vs = [ga0_v, ga1_v]
        sems = [sem0, sem1]
        copies = [None, None]
        qscale = 1.0 / 1023.0

        def transform_of(g):
            row = 2 * g + t
            return [par_v[row, k, :] for k in range(12)]

        def index_pass(g):
            m = transform_of(g)
            idx_v = idx_vs[g]

            def body_idx(i, carry):
                off = i * L
                x = pts_v[0, pl.ds(off, L)]
                y = pts_v[1, pl.ds(off, L)]
                z = pts_v[2, pl.ds(off, L)]
                tx = m[0] * x + m[1] * y + m[2] * z + m[9]
                ty = m[3] * x + m[4] * y + m[5] * z + m[10]
                tz = m[6] * x + m[7] * y + m[8] * z + m[11]
                ux = jnp.clip((tx + 0.5 - 0.015625) * 32.0, 0.0, 31.0)
                uy = jnp.clip((ty + 0.5 - 0.015625) * 32.0, 0.0, 31.0)
                uz = jnp.clip((tz + 0.5 - 0.015625) * 32.0, 0.0, 31.0)
                ix = (ux + 0.5).astype(jnp.int32)
                iy = (uy + 0.5).astype(jnp.int32)
                iz = (uz + 0.5).astype(jnp.int32)
                flat = ix * (GRID * GRID) + iy * GRID + iz
                idx_v[pl.ds(off, L)] = flat + base
                return carry

            lax.fori_loop(0, NGROUPS, body_idx, 0, unroll=4)
            copies[g] = pltpu.async_copy(
                tab_hbm.at[idx_v], ga_vs[g], sems[g])

        def acc_pass(g):
            m = transform_of(g)
            ga_v = ga_vs[g]
            copies[g].wait()

            def body_acc(i, acc):
                off = i * L
                x = pts_v[0, pl.ds(off, L)]
                y = pts_v[1, pl.ds(off, L)]
                z = pts_v[2, pl.ds(off, L)]
                tx = m[0] * x + m[1] * y + m[2] * z + m[9]
                ty = m[3] * x + m[4] * y + m[5] * z + m[10]
                tz = m[6] * x + m[7] * y + m[8] * z + m[11]
                wa = ga_v[pl.ds(off, L)]
                cx = (wa & 1023).astype(jnp.float32) * qscale - 0.5
                cy = (lax.shift_right_logical(wa, 10) & 1023).astype(
                    jnp.float32) * qscale - 0.5
                cz = (lax.shift_right_logical(wa, 20) & 1023).astype(
                    jnp.float32) * qscale - 0.5
                msk = lax.shift_right_logical(wa, 30).astype(jnp.float32)
                dx = tx - cx
                dy = ty - cy
                dz = tz - cz
                return acc + (dx * dx + dy * dy + dz * dz) * msk

            acc = lax.fori_loop(0, NGROUPS, body_acc,
                                jnp.zeros((L,), jnp.float32), unroll=4)
            stage_v[...] = acc
            pltpu.sync_copy(stage_v, out_hbm.at[g, wid])

        index_pass(0)
        index_pass(1)
        acc_pass(0)
        acc_pass(1)

    return sym_loss_kernel


def kernel(points, cp, voxel, plane, quat):
    pts_t = points.transpose(0, 2, 1)

    def quant(a):
        return jnp.round(jnp.clip(a + 0.5, 0.0, 1.0) * 1023.0).astype(
            jnp.int32)

    maskbit = (voxel.reshape(NBATCH, NCELL) < 0.5).astype(jnp.int32)
    tab = (quant(cp[:, :, 0]) | (quant(cp[:, :, 1]) << 10)
           | (quant(cp[:, :, 2]) << 20)
           | (maskbit << 30)).reshape(NBATCH * NCELL)
    par = jnp.broadcast_to(_affine_params(plane, quat)[:, :, None],
                           (4, 12, L))
    parts = _make_kernel()(pts_t, tab, par)
    ref_loss = jnp.sum(parts[0]) / NBATCH
    rot_loss = jnp.sum(parts[1]) / NBATCH
    return (ref_loss, rot_loss)


# speedup vs baseline: 1.6010x; 1.0396x over previous
import functools

import jax
import jax.numpy as jnp
from jax import lax
from jax.experimental import pallas as pl
from jax.experimental.pallas import tpu as pltpu
from jax.experimental.pallas import tpu_sc as plsc

GRID = 32
NCELL = GRID ** 3
NPTS = 8192
NBATCH = 16
L = 16
NGROUPS = NPTS // L
NWORKERS = 32


def _affine_params(plane, quat):
    eye = jnp.eye(3, dtype=jnp.float32)
    n = plane[:, :3]
    n = n / (jnp.linalg.norm(n, axis=1, keepdims=True) + 1e-8)
    d = plane[:, 3:4]
    mp = eye[None] - 2.0 * n[:, :, None] * n[:, None, :]
    op = -2.0 * d * n
    q = quat / (jnp.linalg.norm(quat, axis=1, keepdims=True) + 1e-8)
    w = q[:, 0]
    qv = q[:, 1:]
    s2 = jnp.sum(qv * qv, axis=1)
    zero = jnp.zeros_like(w)
    kx, ky, kz = qv[:, 0], qv[:, 1], qv[:, 2]
    skew = jnp.stack([
        jnp.stack([zero, -kz, ky], axis=1),
        jnp.stack([kz, zero, -kx], axis=1),
        jnp.stack([-ky, kx, zero], axis=1),
    ], axis=1)
    mq = ((1.0 - 2.0 * s2)[:, None, None] * eye[None]
          + 2.0 * qv[:, :, None] * qv[:, None, :]
          + 2.0 * w[:, None, None] * skew)
    oq = jnp.zeros((quat.shape[0], 3), jnp.float32)
    mats = jnp.concatenate([mp, mq], axis=0).reshape(4, 9)
    offs = jnp.concatenate([op, oq], axis=0)
    return jnp.concatenate([mats, offs], axis=1)


def _pack_pair(a, b):
    au = lax.bitcast_convert_type(a.astype(jnp.bfloat16), jnp.uint16)
    bu = lax.bitcast_convert_type(b.astype(jnp.bfloat16), jnp.uint16)
    word = au.astype(jnp.uint32) | (bu.astype(jnp.uint32) << 16)
    return lax.bitcast_convert_type(word, jnp.int32)


@functools.cache
def _make_kernel():
    mesh = plsc.VectorSubcoreMesh(core_axis_name="c", subcore_axis_name="s")

    @functools.partial(
        pl.kernel,
        mesh=mesh,
        out_type=jax.ShapeDtypeStruct((2, NWORKERS, L), jnp.float32),
        scratch_types=[
            pltpu.VMEM((3, NPTS), jnp.float32),
            pltpu.VMEM((NPTS,), jnp.int32),
            pltpu.VMEM((NPTS,), jnp.int32),
            pltpu.VMEM((NPTS,), jnp.int32),
            pltpu.VMEM((NPTS,), jnp.int32),
            pltpu.VMEM((4, 12, L), jnp.float32),
            pltpu.VMEM((L,), jnp.float32),
            pltpu.SemaphoreType.DMA,
            pltpu.SemaphoreType.DMA,
        ],
    )
    def sym_loss_kernel(ptsx_hbm, tab_hbm, par_hbm, out_hbm,
                        pts_v, idx0_v, idx1_v,
                        ga0_v, ga1_v, par_v, stage_v,
                        sem0, sem1):
        c = lax.axis_index("c")
        s = lax.axis_index("s")
        wid = s * 2 + c
        t = wid // NBATCH
        b = wid % NBATCH
        pltpu.sync_copy(ptsx_hbm.at[b], pts_v)
        pltpu.sync_copy(par_hbm, par_v)
        base = b * NCELL
        idx_vs = [idx0_v, idx1_v]
        ga_vs = [ga0_v, ga1_v]
        sems = [sem0, sem1]
        copies = [None, None]
        qscale = 1.0 / 1023.0

        def transform_of(g):
            row = 2 * g + t
            return [par_v[row, k, :] for k in range(12)]

        def index_pass(g):
            m = transform_of(g)
            idx_v = idx_vs[g]

            def body_idx(i, carry):
                off = i * L
                x = pts_v[0, pl.ds(off, L)]
                y = pts_v[1, pl.ds(off, L)]
                z = pts_v[2, pl.ds(off, L)]
                tx = m[0] * x + m[1] * y + m[2] * z + m[9]
                ty = m[3] * x + m[4] * y + m[5] * z + m[10]
                tz = m[6] * x + m[7] * y + m[8] * z + m[11]
                ux = jnp.clip((tx + 0.5 - 0.015625) * 32.0, 0.0, 31.0)
                uy = jnp.clip((ty + 0.5 - 0.015625) * 32.0, 0.0, 31.0)
                uz = jnp.clip((tz + 0.5 - 0.015625) * 32.0, 0.0, 31.0)
                ix = (ux + 0.5).astype(jnp.int32)
                iy = (uy + 0.5).astype(jnp.int32)
                iz = (uz + 0.5).astype(jnp.int32)
                flat = ix * (GRID * GRID) + iy * GRID + iz
                idx_v[pl.ds(off, L)] = flat + base
                return carry

            half = NGROUPS // 2
            lax.fori_loop(0, half, body_idx, 0, unroll=4)
            cp_a = pltpu.async_copy(
                tab_hbm.at[idx_v.at[pl.ds(0, NPTS // 2)]],
                ga_vs[g].at[pl.ds(0, NPTS // 2)], sems[g])
            lax.fori_loop(half, NGROUPS, body_idx, 0, unroll=4)
            cp_b = pltpu.async_copy(
                tab_hbm.at[idx_v.at[pl.ds(NPTS // 2, NPTS // 2)]],
                ga_vs[g].at[pl.ds(NPTS // 2, NPTS // 2)], sems[g])
            copies[g] = (cp_a, cp_b)

        def acc_pass(g):
            m = transform_of(g)
            ga_v = ga_vs[g]
            copies[g][0].wait()
            copies[g][1].wait()

            def body_acc(i, acc):
                off = i * L
                x = pts_v[0, pl.ds(off, L)]
                y = pts_v[1, pl.ds(off, L)]
                z = pts_v[2, pl.ds(off, L)]
                tx = m[0] * x + m[1] * y + m[2] * z + m[9]
                ty = m[3] * x + m[4] * y + m[5] * z + m[10]
                tz = m[6] * x + m[7] * y + m[8] * z + m[11]
                wa = ga_v[pl.ds(off, L)]
                cx = (wa & 1023).astype(jnp.float32) * qscale - 0.5
                cy = (lax.shift_right_logical(wa, 10) & 1023).astype(
                    jnp.float32) * qscale - 0.5
                cz = (lax.shift_right_logical(wa, 20) & 1023).astype(
                    jnp.float32) * qscale - 0.5
                msk = lax.shift_right_logical(wa, 30).astype(jnp.float32)
                dx = tx - cx
                dy = ty - cy
                dz = tz - cz
                return acc + (dx * dx + dy * dy + dz * dz) * msk

            acc = lax.fori_loop(0, NGROUPS, body_acc,
                                jnp.zeros((L,), jnp.float32), unroll=4)
            stage_v[...] = acc
            pltpu.sync_copy(stage_v, out_hbm.at[g, wid])

        index_pass(0)
        index_pass(1)
        acc_pass(0)
        acc_pass(1)

    return sym_loss_kernel


def kernel(points, cp, voxel, plane, quat):
    pts_t = points.transpose(0, 2, 1)

    def quant(a):
        return jnp.round(jnp.clip(a + 0.5, 0.0, 1.0) * 1023.0).astype(
            jnp.int32)

    maskbit = (voxel.reshape(NBATCH, NCELL) < 0.5).astype(jnp.int32)
    tab = (quant(cp[:, :, 0]) | (quant(cp[:, :, 1]) << 10)
           | (quant(cp[:, :, 2]) << 20)
           | (maskbit << 30)).reshape(NBATCH * NCELL)
    par = jnp.broadcast_to(_affine_params(plane, quat)[:, :, None],
                           (4, 12, L))
    parts = _make_kernel()(pts_t, tab, par)
    ref_loss = jnp.sum(parts[0]) / NBATCH
    rot_loss = jnp.sum(parts[1]) / NBATCH
    return (ref_loss, rot_loss)


# speedup vs baseline: 1.6074x; 1.0040x over previous
import functools

import jax
import jax.numpy as jnp
from jax import lax
from jax.experimental import pallas as pl
from jax.experimental.pallas import tpu as pltpu
from jax.experimental.pallas import tpu_sc as plsc

GRID = 32
NCELL = GRID ** 3
NPTS = 8192
NBATCH = 16
L = 16
NGROUPS = NPTS // L
NWORKERS = 32


def _affine_params(plane, quat):
    eye = jnp.eye(3, dtype=jnp.float32)
    n = plane[:, :3]
    n = n / (jnp.linalg.norm(n, axis=1, keepdims=True) + 1e-8)
    d = plane[:, 3:4]
    mp = eye[None] - 2.0 * n[:, :, None] * n[:, None, :]
    op = -2.0 * d * n
    q = quat / (jnp.linalg.norm(quat, axis=1, keepdims=True) + 1e-8)
    w = q[:, 0]
    qv = q[:, 1:]
    s2 = jnp.sum(qv * qv, axis=1)
    zero = jnp.zeros_like(w)
    kx, ky, kz = qv[:, 0], qv[:, 1], qv[:, 2]
    skew = jnp.stack([
        jnp.stack([zero, -kz, ky], axis=1),
        jnp.stack([kz, zero, -kx], axis=1),
        jnp.stack([-ky, kx, zero], axis=1),
    ], axis=1)
    mq = ((1.0 - 2.0 * s2)[:, None, None] * eye[None]
          + 2.0 * qv[:, :, None] * qv[:, None, :]
          + 2.0 * w[:, None, None] * skew)
    oq = jnp.zeros((quat.shape[0], 3), jnp.float32)
    mats = jnp.concatenate([mp, mq], axis=0).reshape(4, 9)
    offs = jnp.concatenate([op, oq], axis=0)
    return jnp.concatenate([mats, offs], axis=1)


def _pack_pair(a, b):
    au = lax.bitcast_convert_type(a.astype(jnp.bfloat16), jnp.uint16)
    bu = lax.bitcast_convert_type(b.astype(jnp.bfloat16), jnp.uint16)
    word = au.astype(jnp.uint32) | (bu.astype(jnp.uint32) << 16)
    return lax.bitcast_convert_type(word, jnp.int32)


@functools.cache
def _make_kernel():
    mesh = plsc.VectorSubcoreMesh(core_axis_name="c", subcore_axis_name="s")

    @functools.partial(
        pl.kernel,
        mesh=mesh,
        out_type=jax.ShapeDtypeStruct((2, NWORKERS, L), jnp.float32),
        scratch_types=[
            pltpu.VMEM((3, NPTS), jnp.float32),
            pltpu.VMEM((NPTS,), jnp.int32),
            pltpu.VMEM((NPTS,), jnp.int32),
            pltpu.VMEM((NPTS,), jnp.int32),
            pltpu.VMEM((NPTS,), jnp.int32),
            pltpu.VMEM((4, 12, L), jnp.float32),
            pltpu.VMEM((L,), jnp.float32),
            pltpu.SemaphoreType.DMA,
            pltpu.SemaphoreType.DMA,
        ],
    )
    def sym_loss_kernel(ptsx_hbm, tab_hbm, par_hbm, out_hbm,
                        pts_v, idx0_v, idx1_v,
                        ga0_v, ga1_v, par_v, stage_v,
                        sem0, sem1):
        c = lax.axis_index("c")
        s = lax.axis_index("s")
        wid = s * 2 + c
        t = wid // NBATCH
        b = wid % NBATCH
        pltpu.sync_copy(ptsx_hbm.at[b], pts_v)
        pltpu.sync_copy(par_hbm, par_v)
        base = b * NCELL
        idx_vs = [idx0_v, idx1_v]
        ga_vs = [ga0_v, ga1_v]
        sems = [sem0, sem1]
        copies = [None, None]
        qscale = 1.0 / 1023.0

        def transform_of(g):
            row = 2 * g + t
            return [par_v[row, k, :] for k in range(12)]

        def index_pass(g):
            m = transform_of(g)
            idx_v = idx_vs[g]

            def body_idx(i, carry):
                off = i * L
                x = pts_v[0, pl.ds(off, L)]
                y = pts_v[1, pl.ds(off, L)]
                z = pts_v[2, pl.ds(off, L)]
                tx = m[0] * x + m[1] * y + m[2] * z + m[9]
                ty = m[3] * x + m[4] * y + m[5] * z + m[10]
                tz = m[6] * x + m[7] * y + m[8] * z + m[11]
                ux = jnp.clip((tx + 0.5 - 0.015625) * 32.0, 0.0, 31.0)
                uy = jnp.clip((ty + 0.5 - 0.015625) * 32.0, 0.0, 31.0)
                uz = jnp.clip((tz + 0.5 - 0.015625) * 32.0, 0.0, 31.0)
                ix = (ux + 0.5).astype(jnp.int32)
                iy = (uy + 0.5).astype(jnp.int32)
                iz = (uz + 0.5).astype(jnp.int32)
                flat = ix * (GRID * GRID) + iy * GRID + iz
                idx_v[pl.ds(off, L)] = flat + base
                return carry

            half = NGROUPS // 2
            lax.fori_loop(0, half, body_idx, 0, unroll=4)
            cp_a = pltpu.async_copy(
                tab_hbm.at[idx_v.at[pl.ds(0, NPTS // 2)]],
                ga_vs[g].at[pl.ds(0, NPTS // 2)], sems[g])
            lax.fori_loop(half, NGROUPS, body_idx, 0, unroll=4)
            cp_b = pltpu.async_copy(
                tab_hbm.at[idx_v.at[pl.ds(NPTS // 2, NPTS // 2)]],
                ga_vs[g].at[pl.ds(NPTS // 2, NPTS // 2)], sems[g])
            copies[g] = (cp_a, cp_b)

        def acc_pass(g):
            m = transform_of(g)
            ga_v = ga_vs[g]

            def body_acc(i, acc):
                off = i * L
                x = pts_v[0, pl.ds(off, L)]
                y = pts_v[1, pl.ds(off, L)]
                z = pts_v[2, pl.ds(off, L)]
                tx = m[0] * x + m[1] * y + m[2] * z + m[9]
                ty = m[3] * x + m[4] * y + m[5] * z + m[10]
                tz = m[6] * x + m[7] * y + m[8] * z + m[11]
                wa = ga_v[pl.ds(off, L)]
                cx = (wa & 1023).astype(jnp.float32) * qscale - 0.5
                cy = (lax.shift_right_logical(wa, 10) & 1023).astype(
                    jnp.float32) * qscale - 0.5
                cz = (lax.shift_right_logical(wa, 20) & 1023).astype(
                    jnp.float32) * qscale - 0.5
                msk = lax.shift_right_logical(wa, 30).astype(jnp.float32)
                dx = tx - cx
                dy = ty - cy
                dz = tz - cz
                return acc + (dx * dx + dy * dy + dz * dz) * msk

            copies[g][0].wait()
            acc = lax.fori_loop(0, NGROUPS // 2, body_acc,
                                jnp.zeros((L,), jnp.float32), unroll=4)
            copies[g][1].wait()
            acc = lax.fori_loop(NGROUPS // 2, NGROUPS, body_acc, acc,
                                unroll=4)
            stage_v[...] = acc
            pltpu.sync_copy(stage_v, out_hbm.at[g, wid])

        index_pass(0)
        index_pass(1)
        acc_pass(0)
        acc_pass(1)

    return sym_loss_kernel


def kernel(points, cp, voxel, plane, quat):
    pts_t = points.transpose(0, 2, 1)

    def quant(a):
        return jnp.round(jnp.clip(a + 0.5, 0.0, 1.0) * 1023.0).astype(
            jnp.int32)

    maskbit = (voxel.reshape(NBATCH, NCELL) < 0.5).astype(jnp.int32)
    tab = (quant(cp[:, :, 0]) | (quant(cp[:, :, 1]) << 10)
           | (quant(cp[:, :, 2]) << 20)
           | (maskbit << 30)).reshape(NBATCH * NCELL)
    par = jnp.broadcast_to(_affine_params(plane, quat)[:, :, None],
                           (4, 12, L))
    parts = _make_kernel()(pts_t, tab, par)
    ref_loss = jnp.sum(parts[0]) / NBATCH
    rot_loss = jnp.sum(parts[1]) / NBATCH
    return (ref_loss, rot_loss)


# speedup vs baseline: 1.6082x; 1.0005x over previous
import functools

import jax
import jax.numpy as jnp
from jax import lax
from jax.experimental import pallas as pl
from jax.experimental.pallas import tpu as pltpu
from jax.experimental.pallas import tpu_sc as plsc

GRID = 32
NCELL = GRID ** 3
NPTS = 8192
NBATCH = 16
L = 16
NGROUPS = NPTS // L
NWORKERS = 32


def _affine_params(plane, quat):
    eye = jnp.eye(3, dtype=jnp.float32)
    n = plane[:, :3]
    n = n / (jnp.linalg.norm(n, axis=1, keepdims=True) + 1e-8)
    d = plane[:, 3:4]
    mp = eye[None] - 2.0 * n[:, :, None] * n[:, None, :]
    op = -2.0 * d * n
    q = quat / (jnp.linalg.norm(quat, axis=1, keepdims=True) + 1e-8)
    w = q[:, 0]
    qv = q[:, 1:]
    s2 = jnp.sum(qv * qv, axis=1)
    zero = jnp.zeros_like(w)
    kx, ky, kz = qv[:, 0], qv[:, 1], qv[:, 2]
    skew = jnp.stack([
        jnp.stack([zero, -kz, ky], axis=1),
        jnp.stack([kz, zero, -kx], axis=1),
        jnp.stack([-ky, kx, zero], axis=1),
    ], axis=1)
    mq = ((1.0 - 2.0 * s2)[:, None, None] * eye[None]
          + 2.0 * qv[:, :, None] * qv[:, None, :]
          + 2.0 * w[:, None, None] * skew)
    oq = jnp.zeros((quat.shape[0], 3), jnp.float32)
    mats = jnp.concatenate([mp, mq], axis=0).reshape(4, 9)
    offs = jnp.concatenate([op, oq], axis=0)
    return jnp.concatenate([mats, offs], axis=1)


@functools.cache
def _make_kernel():
    mesh = plsc.VectorSubcoreMesh(core_axis_name="c", subcore_axis_name="s")

    @functools.partial(
        pl.kernel,
        mesh=mesh,
        out_type=jax.ShapeDtypeStruct((2, NWORKERS, L), jnp.float32),
        scratch_types=[
            pltpu.VMEM((3, NPTS), jnp.float32),
            pltpu.VMEM((NPTS,), jnp.int32),
            pltpu.VMEM((NPTS,), jnp.int32),
            pltpu.VMEM((NPTS,), jnp.int32),
            pltpu.VMEM((NPTS,), jnp.int32),
            pltpu.VMEM((4, 12, L), jnp.float32),
            pltpu.VMEM((L,), jnp.float32),
            pltpu.SemaphoreType.DMA,
            pltpu.SemaphoreType.DMA,
        ],
    )
    def sym_loss_kernel(ptsx_hbm, tab_hbm, par_hbm, out_hbm,
                        pts_v, idx0_v, idx1_v,
                        ga0_v, ga1_v, par_v, stage_v,
                        sem0, sem1):
        c = lax.axis_index("c")
        s = lax.axis_index("s")
        wid = s * 2 + c
        t = wid // NBATCH
        b = wid % NBATCH
        pltpu.sync_copy(ptsx_hbm.at[b], pts_v)
        pltpu.sync_copy(par_hbm, par_v)
        base = b * NCELL
        idx_vs = [idx0_v, idx1_v]
        ga_vs = [ga0_v, ga1_v]
        sems = [sem0, sem1]
        copies = [None, None]
        qscale = 1.0 / 1023.0

        def transform_of(g):
            row = 2 * g + t
            return [par_v[row, k, :] for k in range(12)]

        def index_pass(g):
            m = transform_of(g)
            idx_v = idx_vs[g]

            def body_idx(i, carry):
                off = i * L
                x = pts_v[0, pl.ds(off, L)]
                y = pts_v[1, pl.ds(off, L)]
                z = pts_v[2, pl.ds(off, L)]
                tx = m[0] * x + m[1] * y + m[2] * z + m[9]
                ty = m[3] * x + m[4] * y + m[5] * z + m[10]
                tz = m[6] * x + m[7] * y + m[8] * z + m[11]
                ux = jnp.clip((tx + 0.5 - 0.015625) * 32.0, 0.0, 31.0)
                uy = jnp.clip((ty + 0.5 - 0.015625) * 32.0, 0.0, 31.0)
                uz = jnp.clip((tz + 0.5 - 0.015625) * 32.0, 0.0, 31.0)
                ix = (ux + 0.5).astype(jnp.int32)
                iy = (uy + 0.5).astype(jnp.int32)
                iz = (uz + 0.5).astype(jnp.int32)
                flat = ix * (GRID * GRID) + iy * GRID + iz
                idx_v[pl.ds(off, L)] = flat + base
                return carry

            half = NGROUPS // 2
            lax.fori_loop(0, half, body_idx, 0, unroll=4)
            cp_a = pltpu.async_copy(
                tab_hbm.at[idx_v.at[pl.ds(0, NPTS // 2)]],
                ga_vs[g].at[pl.ds(0, NPTS // 2)], sems[g])
            lax.fori_loop(half, NGROUPS, body_idx, 0, unroll=4)
            cp_b = pltpu.async_copy(
                tab_hbm.at[idx_v.at[pl.ds(NPTS // 2, NPTS // 2)]],
                ga_vs[g].at[pl.ds(NPTS // 2, NPTS // 2)], sems[g])
            copies[g] = (cp_a, cp_b)

        def acc_pass(g):
            m = transform_of(g)
            ga_v = ga_vs[g]

            def body_acc(i, acc):
                off = i * L
                x = pts_v[0, pl.ds(off, L)]
                y = pts_v[1, pl.ds(off, L)]
                z = pts_v[2, pl.ds(off, L)]
                tx = m[0] * x + m[1] * y + m[2] * z + m[9]
                ty = m[3] * x + m[4] * y + m[5] * z + m[10]
                tz = m[6] * x + m[7] * y + m[8] * z + m[11]
                wa = ga_v[pl.ds(off, L)]
                cx = (wa & 1023).astype(jnp.float32) * qscale - 0.5
                cy = (lax.shift_right_logical(wa, 10) & 1023).astype(
                    jnp.float32) * qscale - 0.5
                cz = (lax.shift_right_logical(wa, 20) & 1023).astype(
                    jnp.float32) * qscale - 0.5
                msk = lax.shift_right_logical(wa, 30).astype(jnp.float32)
                dx = tx - cx
                dy = ty - cy
                dz = tz - cz
                return acc + (dx * dx + dy * dy + dz * dz) * msk

            copies[g][0].wait()
            acc = lax.fori_loop(0, NGROUPS // 2, body_acc,
                                jnp.zeros((L,), jnp.float32), unroll=4)
            copies[g][1].wait()
            acc = lax.fori_loop(NGROUPS // 2, NGROUPS, body_acc, acc,
                                unroll=4)
            stage_v[...] = acc
            pltpu.sync_copy(stage_v, out_hbm.at[g, wid])

        index_pass(0)
        index_pass(1)
        acc_pass(0)
        acc_pass(1)

    return sym_loss_kernel


def kernel(points, cp, voxel, plane, quat):
    pts_t = points.transpose(0, 2, 1)

    def quant(a):
        return jnp.round(jnp.clip(a + 0.5, 0.0, 1.0) * 1023.0).astype(
            jnp.int32)

    maskbit = (voxel.reshape(NBATCH, NCELL) < 0.5).astype(jnp.int32)
    tab = (quant(cp[:, :, 0]) | (quant(cp[:, :, 1]) << 10)
           | (quant(cp[:, :, 2]) << 20)
           | (maskbit << 30)).reshape(NBATCH * NCELL)
    par = jnp.broadcast_to(_affine_params(plane, quat)[:, :, None],
                           (4, 12, L))
    parts = _make_kernel()(pts_t, tab, par)
    ref_loss = jnp.sum(parts[0]) / NBATCH
    rot_loss = jnp.sum(parts[1]) / NBATCH
    return (ref_loss, rot_loss)


# speedup vs baseline: 1.6112x; 1.0018x over previous
import functools

import jax
import jax.numpy as jnp
from jax import lax
from jax.experimental import pallas as pl
from jax.experimental.pallas import tpu as pltpu
from jax.experimental.pallas import tpu_sc as plsc

GRID = 32
NCELL = GRID ** 3
NPTS = 8192
NBATCH = 16
L = 16
NGROUPS = NPTS // L
NWORKERS = 32


def _affine_params(plane, quat):
    eye = jnp.eye(3, dtype=jnp.float32)
    n = plane[:, :3]
    n = n / (jnp.linalg.norm(n, axis=1, keepdims=True) + 1e-8)
    d = plane[:, 3:4]
    mp = eye[None] - 2.0 * n[:, :, None] * n[:, None, :]
    op = -2.0 * d * n
    q = quat / (jnp.linalg.norm(quat, axis=1, keepdims=True) + 1e-8)
    w = q[:, 0]
    qv = q[:, 1:]
    s2 = jnp.sum(qv * qv, axis=1)
    zero = jnp.zeros_like(w)
    kx, ky, kz = qv[:, 0], qv[:, 1], qv[:, 2]
    skew = jnp.stack([
        jnp.stack([zero, -kz, ky], axis=1),
        jnp.stack([kz, zero, -kx], axis=1),
        jnp.stack([-ky, kx, zero], axis=1),
    ], axis=1)
    mq = ((1.0 - 2.0 * s2)[:, None, None] * eye[None]
          + 2.0 * qv[:, :, None] * qv[:, None, :]
          + 2.0 * w[:, None, None] * skew)
    oq = jnp.zeros((quat.shape[0], 3), jnp.float32)
    mats = jnp.concatenate([mp, mq], axis=0).reshape(4, 9)
    offs = jnp.concatenate([op, oq], axis=0)
    return jnp.concatenate([mats, offs], axis=1)


@functools.cache
def _make_kernel():
    mesh = plsc.VectorSubcoreMesh(core_axis_name="c", subcore_axis_name="s")

    @functools.partial(
        pl.kernel,
        mesh=mesh,
        out_type=jax.ShapeDtypeStruct((2, NWORKERS, L), jnp.float32),
        scratch_types=[
            pltpu.VMEM((3, NPTS), jnp.float32),
            pltpu.VMEM((NPTS,), jnp.int32),
            pltpu.VMEM((NPTS,), jnp.int32),
            pltpu.VMEM((NPTS,), jnp.int32),
            pltpu.VMEM((NPTS,), jnp.int32),
            pltpu.VMEM((4, 12, L), jnp.float32),
            pltpu.VMEM((L,), jnp.float32),
            pltpu.SemaphoreType.DMA,
            pltpu.SemaphoreType.DMA,
        ],
    )
    def sym_loss_kernel(ptsx_hbm, tab_hbm, par_hbm, out_hbm,
                        pts_v, idx0_v, idx1_v,
                        ga0_v, ga1_v, par_v, stage_v,
                        sem0, sem1):
        c = lax.axis_index("c")
        s = lax.axis_index("s")
        wid = s * 2 + c
        t = wid // NBATCH
        b = wid % NBATCH
        pltpu.sync_copy(ptsx_hbm.at[b], pts_v)
        pltpu.sync_copy(par_hbm, par_v)
        base = b * NCELL
        idx_vs = [idx0_v, idx1_v]
        ga_vs = [ga0_v, ga1_v]
        sems = [sem0, sem1]
        copies = [None, None]
        qscale = 1.0 / 1023.0

        def transform_of(g):
            row = 2 * g + t
            return [par_v[row, k, :] for k in range(12)]

        def index_pass(g):
            m = transform_of(g)
            idx_v = idx_vs[g]

            def body_idx(i, carry):
                off = i * L
                x = pts_v[0, pl.ds(off, L)]
                y = pts_v[1, pl.ds(off, L)]
                z = pts_v[2, pl.ds(off, L)]
                tx = m[0] * x + m[1] * y + m[2] * z + m[9]
                ty = m[3] * x + m[4] * y + m[5] * z + m[10]
                tz = m[6] * x + m[7] * y + m[8] * z + m[11]
                ux = jnp.clip((tx + 0.5 - 0.015625) * 32.0, 0.0, 31.0)
                uy = jnp.clip((ty + 0.5 - 0.015625) * 32.0, 0.0, 31.0)
                uz = jnp.clip((tz + 0.5 - 0.015625) * 32.0, 0.0, 31.0)
                ix = (ux + 0.5).astype(jnp.int32)
                iy = (uy + 0.5).astype(jnp.int32)
                iz = (uz + 0.5).astype(jnp.int32)
                flat = ix * (GRID * GRID) + iy * GRID + iz
                idx_v[pl.ds(off, L)] = flat + base
                return carry

            quarter = NGROUPS // 4
            qpts = NPTS // 4
            cps = []
            for q in range(4):
                lax.fori_loop(q * quarter, (q + 1) * quarter, body_idx, 0,
                              unroll=4)
                cps.append(pltpu.async_copy(
                    tab_hbm.at[idx_v.at[pl.ds(q * qpts, qpts)]],
                    ga_vs[g].at[pl.ds(q * qpts, qpts)], sems[g]))
            copies[g] = tuple(cps)

        def acc_pass(g):
            m = transform_of(g)
            ga_v = ga_vs[g]

            def body_acc(i, acc):
                off = i * L
                x = pts_v[0, pl.ds(off, L)]
                y = pts_v[1, pl.ds(off, L)]
                z = pts_v[2, pl.ds(off, L)]
                tx = m[0] * x + m[1] * y + m[2] * z + m[9]
                ty = m[3] * x + m[4] * y + m[5] * z + m[10]
                tz = m[6] * x + m[7] * y + m[8] * z + m[11]
                wa = ga_v[pl.ds(off, L)]
                cx = (wa & 1023).astype(jnp.float32) * qscale - 0.5
                cy = (lax.shift_right_logical(wa, 10) & 1023).astype(
                    jnp.float32) * qscale - 0.5
                cz = (lax.shift_right_logical(wa, 20) & 1023).astype(
                    jnp.float32) * qscale - 0.5
                msk = lax.shift_right_logical(wa, 30).astype(jnp.float32)
                dx = tx - cx
                dy = ty - cy
                dz = tz - cz
                return acc + (dx * dx + dy * dy + dz * dz) * msk

            quarter = NGROUPS // 4
            acc = jnp.zeros((L,), jnp.float32)
            for q in range(4):
                copies[g][q].wait()
                acc = lax.fori_loop(q * quarter, (q + 1) * quarter,
                                    body_acc, acc, unroll=4)
            stage_v[...] = acc
            pltpu.sync_copy(stage_v, out_hbm.at[g, wid])

        index_pass(0)
        index_pass(1)
        acc_pass(0)
        acc_pass(1)

    return sym_loss_kernel


def kernel(points, cp, voxel, plane, quat):
    pts_t = points.transpose(0, 2, 1)

    def quant(a):
        return jnp.round(jnp.clip(a + 0.5, 0.0, 1.0) * 1023.0).astype(
            jnp.int32)

    maskbit = (voxel.reshape(NBATCH, NCELL) < 0.5).astype(jnp.int32)
    tab = (quant(cp[:, :, 0]) | (quant(cp[:, :, 1]) << 10)
           | (quant(cp[:, :, 2]) << 20)
           | (maskbit << 30)).reshape(NBATCH * NCELL)
    par = jnp.broadcast_to(_affine_params(plane, quat)[:, :, None],
                           (4, 12, L))
    parts = _make_kernel()(pts_t, tab, par)
    ref_loss = jnp.sum(parts[0]) / NBATCH
    rot_loss = jnp.sum(parts[1]) / NBATCH
    return (ref_loss, rot_loss)
